# baseline (speedup 1.0000x reference)
import functools

import jax
import jax.numpy as jnp
from jax import lax
from jax.experimental import pallas as pl
from jax.experimental.pallas import tpu as pltpu
from jax.experimental.pallas import tpu_sc as plsc

N = 10000
FIN = 128
HID = 64
NCLS = 40
E = 320000

NC = 2
NS = 16
NW = NC * NS

CH = 128
NCHA = 80
EPW = NCHA * CH
EPAD = EPW * NW
NCHD = -(-E // (NS * CH))
EPAD_D = NCHD * CH * NS
MPAD = 10240
VPT = MPAD // NS
RPT = MPAD // NS
ZR = 128

_sc_mesh = plsc.VectorSubcoreMesh(core_axis_name="c", subcore_axis_name="s")


@functools.partial(
    pl.kernel,
    out_type=jax.ShapeDtypeStruct((MPAD,), jnp.float32),
    mesh=_sc_mesh,
    compiler_params=pltpu.CompilerParams(use_tc_tiling_on_sc=False),
    scratch_types=[
        pltpu.VMEM((NCHD, CH), jnp.int32),
        pltpu.VMEM((CH,), jnp.float32),
        pltpu.VMEM((VPT,), jnp.float32),
        pltpu.VMEM_SHARED((MPAD,), jnp.float32),
        pltpu.SemaphoreType.DMA,
    ],
)
def _sc_deg(dst_hbm, out_hbm, didx, ones, buf, dacc, sem):
    c = lax.axis_index("c")
    s = lax.axis_index("s")

    @pl.when(c == 0)
    def _():
        for j in range(CH // 16):
            ones[pl.ds(j * 16, 16)] = jnp.full((16,), 1.0, jnp.float32)

        def zero(j, carry):
            buf[pl.ds(j * 16, 16)] = jnp.zeros((16,), jnp.float32)
            return carry

        lax.fori_loop(0, VPT // 16, zero, 0)
        pltpu.sync_copy(buf, dacc.at[pl.ds(s * VPT, VPT)])
        pltpu.sync_copy(dst_hbm.at[s], didx)
        plsc.subcore_barrier()

        def grp(g, carry):
            base = g * 8
            for k in range(8):
                pltpu.async_copy(ones, dacc.at[didx.at[base + k]], sem, add=True)
            for k in range(8):
                pltpu.make_async_copy(ones, dacc.at[didx.at[base]], sem).wait()
            return carry

        ngrp = NCHD // 8
        lax.fori_loop(0, ngrp, grp, 0)
        for k in range(NCHD - ngrp * 8):
            pltpu.sync_copy(ones, dacc.at[didx.at[ngrp * 8 + k]], add=True)
        plsc.subcore_barrier()
        pltpu.sync_copy(dacc.at[pl.ds(s * VPT, VPT)],
                        out_hbm.at[pl.ds(s * VPT, VPT)])


@functools.partial(
    pl.kernel,
    out_type=jax.ShapeDtypeStruct((NC * MPAD, HID), jnp.float32),
    mesh=_sc_mesh,
    compiler_params=pltpu.CompilerParams(use_tc_tiling_on_sc=False),
    scratch_types=[
        pltpu.VMEM((NCHA, CH), jnp.int32),
        pltpu.VMEM((NCHA, CH), jnp.int32),
        pltpu.VMEM((CH, HID), jnp.float32),
        pltpu.VMEM((CH, HID), jnp.float32),
        pltpu.VMEM((ZR, HID), jnp.float32),
        pltpu.VMEM_SHARED((MPAD, HID), jnp.float32),
        pltpu.VMEM_SHARED((MPAD, HID), jnp.float32),
        [pltpu.SemaphoreType.DMA] * 2,
        [pltpu.SemaphoreType.DMA] * 2,
    ],
)
def _sc_agg(mp_hbm, src_hbm, dst_hbm, out_hbm, sidx, didx, rows0, rows1, zbuf,
            mps, acc, gsems, ssems):
    c = lax.axis_index("c")
    s = lax.axis_index("s")
    w = c * NS + s
    bufs = (rows0, rows1)

    def zero(t, carry):
        for j in range(HID // 16):
            zbuf[t, pl.ds(j * 16, 16)] = jnp.zeros((16,), jnp.float32)
        return carry

    lax.fori_loop(0, ZR, zero, 0)
    pltpu.sync_copy(src_hbm.at[w], sidx)
    pltpu.sync_copy(dst_hbm.at[w], didx)
    pltpu.sync_copy(mp_hbm.at[pl.ds(s * RPT, RPT)], mps.at[pl.ds(s * RPT, RPT)])
    for k in range(RPT // ZR):
        pltpu.sync_copy(zbuf, acc.at[pl.ds(s * RPT + k * ZR, ZR)])
    plsc.subcore_barrier()

    def g_start(i, b, tab):
        pltpu.async_copy(tab.at[sidx.at[i]], bufs[b], gsems[b])

    def g_wait(i, b, tab):
        pltpu.make_async_copy(tab.at[sidx.at[i]], bufs[b], gsems[b]).wait()

    def s_start(i, b):
        pltpu.async_copy(bufs[b], acc.at[didx.at[i]], ssems[b], add=True)

    def s_wait(i, b):
        pltpu.make_async_copy(bufs[b], acc.at[didx.at[i]], ssems[b]).wait()

    g_start(0, 0, mps)
    g_wait(0, 0, mps)
    s_start(0, 0)
    g_start(1, 1, mp_hbm)

    def body2(j, carry):
        for k in range(2):
            i = 2 * j + 1 + k
            b = (1 + k) % 2
            tab = mp_hbm if k == 0 else mps
            ntab = mps if k == 0 else mp_hbm
            g_wait(i, b, tab)
            s_start(i, b)
            s_wait(i - 1, 1 - b)
            g_start(i + 1, 1 - b, ntab)
        return carry

    lax.fori_loop(0, (NCHA - 2) // 2, body2, 0)
    g_wait(NCHA - 1, 1, mp_hbm)
    s_start(NCHA - 1, 1)
    s_wait(NCHA - 2, 0)
    s_wait(NCHA - 1, 1)

    plsc.subcore_barrier()
    pltpu.sync_copy(acc.at[pl.ds(s * RPT, RPT)],
                    out_hbm.at[pl.ds(c * MPAD + s * RPT, RPT)])


def _tc_first_body(x_ref, w_ref, deg_ref, o_ref):
    dinv = lax.rsqrt(deg_ref[...] + 1.0)
    o_ref[...] = (
        jnp.dot(x_ref[...], w_ref[...], preferred_element_type=jnp.float32) * dinv
    )


def _tc_mid_body(p_ref, mp_ref, deg_ref, b_ref, w_ref, o_ref):
    dinv = lax.rsqrt(deg_ref[...] + 1.0)
    t = (p_ref[:MPAD] + p_ref[MPAD:] + mp_ref[...]) * dinv + b_ref[...]
    h = jnp.maximum(t, 0.0)
    o_ref[...] = (
        jnp.dot(h, w_ref[...], preferred_element_type=jnp.float32) * dinv
    )


def _tc_final_body(p_ref, mp_ref, deg_ref, b3_ref, wl1_ref, bl1_ref, wl2_ref,
                   bl2_ref, o_ref):
    dinv = lax.rsqrt(deg_ref[...] + 1.0)
    t = (p_ref[:MPAD] + p_ref[MPAD:] + mp_ref[...]) * dinv + b3_ref[...]
    h = jnp.maximum(t, 0.0)
    h = jnp.maximum(
        jnp.dot(h, wl1_ref[...], preferred_element_type=jnp.float32) + bl1_ref[...],
        0.0,
    )
    lg = jnp.dot(h, wl2_ref[...], preferred_element_type=jnp.float32) + bl2_ref[...]
    m = jnp.max(lg, axis=-1, keepdims=True)
    lse = jnp.log(jnp.sum(jnp.exp(lg - m), axis=-1, keepdims=True)) + m
    o_ref[...] = lg - lse


def kernel(x, edge_index, W1, b1, W2, b2, W3, b3, Wl1, bl1, Wl2, bl2):
    src = edge_index[0]
    dst = edge_index[1]
    srcp = jnp.concatenate(
        [src, jnp.zeros((EPAD - E,), jnp.int32)]).reshape(NW, NCHA, CH)
    dstp = jnp.concatenate(
        [dst, jnp.full((EPAD - E,), N, jnp.int32)]).reshape(NW, NCHA, CH)
    dstd = jnp.concatenate(
        [dst, jnp.full((EPAD_D - E,), N, jnp.int32)]).reshape(NS, NCHD, CH)

    deg = _sc_deg(dstd).reshape(MPAD, 1)
    xp = jnp.pad(x, ((0, MPAD - N), (0, 0)))

    mp1 = pl.pallas_call(
        _tc_first_body,
        out_shape=jax.ShapeDtypeStruct((MPAD, HID), jnp.float32),
    )(xp, W1, deg)
    p1 = _sc_agg(mp1, srcp, dstp)
    mp2 = pl.pallas_call(
        _tc_mid_body,
        out_shape=jax.ShapeDtypeStruct((MPAD, HID), jnp.float32),
    )(p1, mp1, deg, b1.reshape(1, HID), W2)
    p2 = _sc_agg(mp2, srcp, dstp)
    mp3 = pl.pallas_call(
        _tc_mid_body,
        out_shape=jax.ShapeDtypeStruct((MPAD, HID), jnp.float32),
    )(p2, mp2, deg, b2.reshape(1, HID), W3)
    p3 = _sc_agg(mp3, srcp, dstp)
    out = pl.pallas_call(
        _tc_final_body,
        out_shape=jax.ShapeDtypeStruct((MPAD, NCLS), jnp.float32),
    )(p3, mp3, deg, b3.reshape(1, HID), Wl1, bl1.reshape(1, HID),
      Wl2, bl2.reshape(1, NCLS))
    return out[:N]


# speedup vs baseline: 1.8010x; 1.8010x over previous
import functools

import jax
import jax.numpy as jnp
from jax import lax
from jax.experimental import pallas as pl
from jax.experimental.pallas import tpu as pltpu
from jax.experimental.pallas import tpu_sc as plsc

N = 10000
FIN = 128
HID = 64
NCLS = 40
E = 320000

NC = 2
NS = 16
NW = NC * NS

CH = 128
NCHA = 80
EPW = NCHA * CH
EPAD = EPW * NW
MPAD = 10240
VPT = MPAD // NS
RPT = MPAD // NS
ZR = 128

_sc_mesh = plsc.VectorSubcoreMesh(core_axis_name="c", subcore_axis_name="s")


@functools.partial(
    pl.kernel,
    out_type=jax.ShapeDtypeStruct((NC * MPAD,), jnp.float32),
    mesh=_sc_mesh,
    compiler_params=pltpu.CompilerParams(use_tc_tiling_on_sc=False),
    scratch_types=[
        pltpu.VMEM((NCHA, CH), jnp.int32),
        pltpu.VMEM((CH,), jnp.float32),
        pltpu.VMEM((VPT,), jnp.float32),
        pltpu.VMEM_SHARED((MPAD,), jnp.float32),
        pltpu.SemaphoreType.DMA,
    ],
)
def _sc_deg(dst_hbm, out_hbm, didx, ones, buf, dacc, sem):
    c = lax.axis_index("c")
    s = lax.axis_index("s")
    w = c * NS + s

    for j in range(CH // 16):
        ones[pl.ds(j * 16, 16)] = jnp.full((16,), 1.0, jnp.float32)

    def zero(j, carry):
        buf[pl.ds(j * 16, 16)] = jnp.zeros((16,), jnp.float32)
        return carry

    lax.fori_loop(0, VPT // 16, zero, 0)
    pltpu.sync_copy(buf, dacc.at[pl.ds(s * VPT, VPT)])
    pltpu.sync_copy(dst_hbm.at[w], didx)
    plsc.subcore_barrier()

    def grp(g, carry):
        base = g * 8
        for k in range(8):
            pltpu.async_copy(ones, dacc.at[didx.at[base + k]], sem, add=True)
        for k in range(8):
            pltpu.make_async_copy(ones, dacc.at[didx.at[base]], sem).wait()
        return carry

    lax.fori_loop(0, NCHA // 8, grp, 0)
    plsc.subcore_barrier()
    pltpu.sync_copy(dacc.at[pl.ds(s * VPT, VPT)],
                    out_hbm.at[pl.ds(c * MPAD + s * VPT, VPT)])


@functools.partial(
    pl.kernel,
    out_type=jax.ShapeDtypeStruct((NC * MPAD, HID), jnp.float32),
    mesh=_sc_mesh,
    compiler_params=pltpu.CompilerParams(use_tc_tiling_on_sc=False),
    scratch_types=[
        pltpu.VMEM((NCHA, CH), jnp.int32),
        pltpu.VMEM((NCHA, CH), jnp.int32),
        pltpu.VMEM((CH, HID), jnp.float32),
        pltpu.VMEM((CH, HID), jnp.float32),
        pltpu.VMEM((ZR, HID), jnp.float32),
        pltpu.VMEM_SHARED((MPAD, HID), jnp.float32),
        pltpu.VMEM_SHARED((MPAD, HID), jnp.float32),
        [pltpu.SemaphoreType.DMA] * 2,
        [pltpu.SemaphoreType.DMA] * 2,
    ],
)
def _sc_agg(mp_hbm, src_hbm, dst_hbm, out_hbm, sidx, didx, rows0, rows1, zbuf,
            mps, acc, gsems, ssems):
    c = lax.axis_index("c")
    s = lax.axis_index("s")
    w = c * NS + s
    bufs = (rows0, rows1)

    def zero(t, carry):
        for j in range(HID // 16):
            zbuf[t, pl.ds(j * 16, 16)] = jnp.zeros((16,), jnp.float32)
        return carry

    pltpu.async_copy(src_hbm.at[w], sidx, gsems[0])
    pltpu.async_copy(dst_hbm.at[w], didx, gsems[1])
    pltpu.async_copy(mp_hbm.at[pl.ds(s * RPT, RPT)],
                     mps.at[pl.ds(s * RPT, RPT)], ssems[0])
    lax.fori_loop(0, ZR, zero, 0)
    for k in range(RPT // ZR):
        pltpu.async_copy(zbuf, acc.at[pl.ds(s * RPT + k * ZR, ZR)], ssems[1])
    pltpu.make_async_copy(src_hbm.at[w], sidx, gsems[0]).wait()
    pltpu.make_async_copy(dst_hbm.at[w], didx, gsems[1]).wait()
    pltpu.make_async_copy(mp_hbm.at[pl.ds(s * RPT, RPT)],
                          mps.at[pl.ds(s * RPT, RPT)], ssems[0]).wait()
    for k in range(RPT // ZR):
        pltpu.make_async_copy(zbuf, acc.at[pl.ds(s * RPT + k * ZR, ZR)],
                              ssems[1]).wait()
    plsc.subcore_barrier()

    def g_start(i, b):
        pltpu.async_copy(mps.at[sidx.at[i]], bufs[b], gsems[b])

    def g_wait(i, b):
        pltpu.make_async_copy(mps.at[sidx.at[i]], bufs[b], gsems[b]).wait()

    def s_start(i, b):
        pltpu.async_copy(bufs[b], acc.at[didx.at[i]], ssems[b], add=True)

    def s_wait(i, b):
        pltpu.make_async_copy(bufs[b], acc.at[didx.at[i]], ssems[b]).wait()

    g_start(0, 0)
    g_wait(0, 0)
    s_start(0, 0)
    g_start(1, 1)

    def body2(j, carry):
        for k in range(2):
            i = 2 * j + 1 + k
            b = (1 + k) % 2
            g_wait(i, b)
            s_start(i, b)
            s_wait(i - 1, 1 - b)
            g_start(i + 1, 1 - b)
        return carry

    lax.fori_loop(0, (NCHA - 2) // 2, body2, 0)
    g_wait(NCHA - 1, 1)
    s_start(NCHA - 1, 1)
    s_wait(NCHA - 2, 0)
    s_wait(NCHA - 1, 1)

    plsc.subcore_barrier()
    pltpu.sync_copy(acc.at[pl.ds(s * RPT, RPT)],
                    out_hbm.at[pl.ds(c * MPAD + s * RPT, RPT)])


def _tc_first_body(x_ref, w_ref, deg_ref, o_ref):
    dinv = lax.rsqrt(deg_ref[:N] + deg_ref[MPAD:MPAD + N] + 1.0)
    o_ref[:N] = (
        jnp.dot(x_ref[...], w_ref[...], preferred_element_type=jnp.float32) * dinv
    )
    o_ref[N:] = jnp.zeros((MPAD - N, HID), jnp.float32)


def _tc_mid_body(p_ref, mp_ref, deg_ref, b_ref, w_ref, o_ref):
    dinv = lax.rsqrt(deg_ref[:MPAD] + deg_ref[MPAD:] + 1.0)
    t = (p_ref[:MPAD] + p_ref[MPAD:] + mp_ref[...]) * dinv + b_ref[...]
    h = jnp.maximum(t, 0.0)
    o_ref[...] = (
        jnp.dot(h, w_ref[...], preferred_element_type=jnp.float32) * dinv
    )


def _tc_final_body(p_ref, mp_ref, deg_ref, b3_ref, wl1_ref, bl1_ref, wl2_ref,
                   bl2_ref, o_ref):
    dinv = lax.rsqrt(deg_ref[:MPAD] + deg_ref[MPAD:] + 1.0)
    t = (p_ref[:MPAD] + p_ref[MPAD:] + mp_ref[...]) * dinv + b3_ref[...]
    h = jnp.maximum(t, 0.0)
    h = jnp.maximum(
        jnp.dot(h, wl1_ref[...], preferred_element_type=jnp.float32) + bl1_ref[...],
        0.0,
    )
    lg = jnp.dot(h, wl2_ref[...], preferred_element_type=jnp.float32) + bl2_ref[...]
    m = jnp.max(lg, axis=-1, keepdims=True)
    lse = jnp.log(jnp.sum(jnp.exp(lg - m), axis=-1, keepdims=True)) + m
    o_ref[...] = lg - lse


def kernel(x, edge_index, W1, b1, W2, b2, W3, b3, Wl1, bl1, Wl2, bl2):
    src = edge_index[0]
    dst = edge_index[1]
    srcp = jnp.concatenate(
        [src, jnp.zeros((EPAD - E,), jnp.int32)]).reshape(NW, NCHA, CH)
    dstp = jnp.concatenate(
        [dst, jnp.full((EPAD - E,), N, jnp.int32)]).reshape(NW, NCHA, CH)
    deg = _sc_deg(dstp).reshape(NC * MPAD, 1)

    mp1 = pl.pallas_call(
        _tc_first_body,
        out_shape=jax.ShapeDtypeStruct((MPAD, HID), jnp.float32),
    )(x, W1, deg)
    p1 = _sc_agg(mp1, srcp, dstp)
    mp2 = pl.pallas_call(
        _tc_mid_body,
        out_shape=jax.ShapeDtypeStruct((MPAD, HID), jnp.float32),
    )(p1, mp1, deg, b1.reshape(1, HID), W2)
    p2 = _sc_agg(mp2, srcp, dstp)
    mp3 = pl.pallas_call(
        _tc_mid_body,
        out_shape=jax.ShapeDtypeStruct((MPAD, HID), jnp.float32),
    )(p2, mp2, deg, b2.reshape(1, HID), W3)
    p3 = _sc_agg(mp3, srcp, dstp)
    out = pl.pallas_call(
        _tc_final_body,
        out_shape=jax.ShapeDtypeStruct((MPAD, NCLS), jnp.float32),
    )(p3, mp3, deg, b3.reshape(1, HID), Wl1, bl1.reshape(1, HID),
      Wl2, bl2.reshape(1, NCLS))
    return out[:N]
